# baseline (speedup 1.0000x reference)
import functools

import jax
import jax.numpy as jnp
from jax import lax
from jax.experimental import pallas as pl
from jax.experimental.pallas import tpu as pltpu
from jax.experimental.pallas import tpu_sc as plsc

N_NODES = 10000
N_EDGES = 320000
D_FEAT = 128

NC = 2
NS = 16
HALF = N_NODES // NC
CH = 80
NCH = 250
E_PER_S = NCH * CH
N_LOC = 5120
RPT = N_LOC // NS
ZB = 64
NBUF = 2

_sc_mesh = plsc.VectorSubcoreMesh(core_axis_name="c", subcore_axis_name="s")


def _edge_body(hnorm_hbm, srcr_hbm, dstr_hbm, out_hbm,
               src_v, dst_v, *rest):
    rows = rest[:NBUF]
    r0 = rows[0]
    acc = rest[NBUF]
    gsems = rest[NBUF + 1:NBUF + 1 + NBUF]
    ssems = rest[NBUF + 1 + NBUF:]
    cid = lax.axis_index("c")
    sid = lax.axis_index("s")

    zv = jnp.zeros((16,), jnp.float32)

    def zbody(i, _):
        for k in range(D_FEAT // 16):
            r0[i, pl.ds(k * 16, 16)] = zv
        return 0

    lax.fori_loop(0, ZB, zbody, 0)
    for k in range(RPT // ZB):
        pltpu.sync_copy(r0.at[pl.ds(0, ZB)],
                        acc.at[pl.ds(sid * RPT + k * ZB, ZB)])

    pltpu.sync_copy(srcr_hbm.at[sid], src_v)
    pltpu.sync_copy(dstr_hbm.at[cid, sid], dst_v)
    plsc.subcore_barrier()

    for k in range(NBUF):
        pltpu.async_copy(hnorm_hbm.at[src_v.at[k]], rows[k], gsems[k])

    def body(jj, _):
        for k in range(NBUF):
            j = NBUF * jj + k
            pltpu.make_async_copy(
                hnorm_hbm.at[src_v.at[j]], rows[k], gsems[k]).wait()
            pltpu.async_copy(rows[k], acc.at[dst_v.at[j]], ssems[k],
                             add=True)
            kp = (k - 1) % NBUF
            jn = j - 1 + NBUF

            @pl.when(jnp.logical_and(j >= 1, jn < NCH))
            def _():
                pltpu.make_async_copy(
                    rows[kp], acc.at[dst_v.at[jn - NBUF]], ssems[kp]).wait()
                pltpu.async_copy(
                    hnorm_hbm.at[src_v.at[jn]], rows[kp], gsems[kp])

        return 0

    lax.fori_loop(0, NCH // NBUF, body, 0)

    for k in range(NBUF):
        j = NCH - NBUF + k
        pltpu.make_async_copy(rows[k], acc.at[dst_v.at[j]], ssems[k]).wait()

    plsc.subcore_barrier()
    pltpu.sync_copy(acc.at[pl.ds(sid * RPT, RPT)],
                    out_hbm.at[cid, pl.ds(sid * RPT, RPT)])


_edge_call = functools.partial(
    pl.kernel,
    out_type=jax.ShapeDtypeStruct((NC, N_LOC, D_FEAT), jnp.float32),
    mesh=_sc_mesh,
    scratch_types=[
        pltpu.VMEM((NCH, CH), jnp.int32),
        pltpu.VMEM((NCH, CH), jnp.int32),
    ] + [pltpu.VMEM((CH, D_FEAT), jnp.float32)] * NBUF + [
        pltpu.VMEM_SHARED((N_LOC, D_FEAT), jnp.float32),
    ] + [pltpu.SemaphoreType.DMA] * (2 * NBUF),
)(_edge_body)


ROWS_BLK = 400
N_BLKS = N_NODES // ROWS_BLK


def _hnorm_body(h_ref, d_ref, o_ref):
    o_ref[...] = h_ref[...] / d_ref[...]


def _hnorm(hidden, degree):
    return pl.pallas_call(
        _hnorm_body,
        out_shape=jax.ShapeDtypeStruct((N_NODES, D_FEAT), jnp.float32),
        grid=(N_BLKS,),
        in_specs=[
            pl.BlockSpec((ROWS_BLK, D_FEAT), lambda i: (i, 0)),
            pl.BlockSpec((ROWS_BLK, 1), lambda i: (i, 0)),
        ],
        out_specs=pl.BlockSpec((ROWS_BLK, D_FEAT), lambda i: (i, 0)),
    )(hidden, degree)


APPLY_BLK = 200
APPLY_PER_CORE = HALF // APPLY_BLK
N_APPLY_BLKS = N_NODES // APPLY_BLK


def _apply_body(stack_ref, p_ref, h_ref, wt_ref, b_ref, o_ref):
    red = p_ref[0] + h_ref[...]
    o_ref[:, :D_FEAT] = stack_ref[...]
    o_ref[:, D_FEAT:] = (
        jnp.dot(red, wt_ref[...], preferred_element_type=jnp.float32)
        + b_ref[...]
    )


def _apply(stack, partials, hidden, wt, b2):
    return pl.pallas_call(
        _apply_body,
        out_shape=jax.ShapeDtypeStruct((N_NODES, 2 * D_FEAT), jnp.float32),
        grid=(N_APPLY_BLKS,),
        in_specs=[
            pl.BlockSpec((APPLY_BLK, D_FEAT), lambda i: (i, 0)),
            pl.BlockSpec(
                (1, APPLY_BLK, D_FEAT),
                lambda i: (i // APPLY_PER_CORE, i % APPLY_PER_CORE, 0),
            ),
            pl.BlockSpec((APPLY_BLK, D_FEAT), lambda i: (i, 0)),
            pl.BlockSpec((D_FEAT, D_FEAT), lambda i: (0, 0)),
            pl.BlockSpec((1, D_FEAT), lambda i: (0, 0)),
        ],
        out_specs=pl.BlockSpec((APPLY_BLK, 2 * D_FEAT), lambda i: (i, 0)),
    )(stack, partials, hidden, wt, b2)


def kernel(hidden, degree, stack, W, b, edge_index):
    hnorm = _hnorm(hidden, degree)
    src = edge_index[0]
    dst = edge_index[1]
    trash = HALF + jnp.arange(N_EDGES, dtype=jnp.int32) % (N_LOC - HALF)
    d0 = jnp.where(dst < HALF, dst, trash)
    d1 = jnp.where(dst >= HALF, dst - HALF, trash)
    srcr = src.reshape(NS, NCH, CH)
    dstr = jnp.stack([d0, d1]).reshape(NC, NS, NCH, CH)
    partials = _edge_call(hnorm, srcr, dstr)
    return _apply(stack, partials, hidden, W.T, b.reshape(1, D_FEAT))


# speedup vs baseline: 1.3682x; 1.3682x over previous
import functools

import jax
import jax.numpy as jnp
from jax import lax
from jax.experimental import pallas as pl
from jax.experimental.pallas import tpu as pltpu
from jax.experimental.pallas import tpu_sc as plsc

N_NODES = 10000
N_EDGES = 320000
D_FEAT = 128

NC = 2
NS = 16
HALF = N_NODES // NC
CH = 80
NCH = 250
E_PER_S = NCH * CH
N_LOC = 5120
RPT = N_LOC // NS
ZB = 64
NBUF = 2

_sc_mesh = plsc.VectorSubcoreMesh(core_axis_name="c", subcore_axis_name="s")


def _edge_body(hnorm_hbm, srcr_hbm, dstr_hbm, out_hbm,
               src_v, dst_v, *rest):
    rows = rest[:NBUF]
    r0 = rows[0]
    acc = rest[NBUF]
    gsems = rest[NBUF + 1:]
    cid = lax.axis_index("c")
    sid = lax.axis_index("s")

    zv = jnp.zeros((16,), jnp.float32)

    def zbody(i, _):
        for k in range(D_FEAT // 16):
            r0[i, pl.ds(k * 16, 16)] = zv
        return 0

    lax.fori_loop(0, ZB, zbody, 0)
    for k in range(RPT // ZB):
        pltpu.sync_copy(r0.at[pl.ds(0, ZB)],
                        acc.at[pl.ds(sid * RPT + k * ZB, ZB)])

    pltpu.sync_copy(srcr_hbm.at[sid], src_v)
    pltpu.sync_copy(dstr_hbm.at[cid, sid], dst_v)
    plsc.subcore_barrier()

    for k in range(NBUF):
        pltpu.async_copy(hnorm_hbm.at[src_v.at[k]], rows[k], gsems[k])

    def body(jj, _):
        for k in range(NBUF):
            j = NBUF * jj + k
            pltpu.make_async_copy(
                hnorm_hbm.at[src_v.at[j]], rows[k], gsems[k]).wait()
            pltpu.sync_copy(rows[k], acc.at[dst_v.at[j]], add=True)

            @pl.when(j + NBUF < NCH)
            def _():
                pltpu.async_copy(
                    hnorm_hbm.at[src_v.at[j + NBUF]], rows[k], gsems[k])

        return 0

    lax.fori_loop(0, NCH // NBUF, body, 0)

    plsc.subcore_barrier()
    pltpu.sync_copy(acc.at[pl.ds(sid * RPT, RPT)],
                    out_hbm.at[cid, pl.ds(sid * RPT, RPT)])


_edge_call = functools.partial(
    pl.kernel,
    out_type=jax.ShapeDtypeStruct((NC, N_LOC, D_FEAT), jnp.float32),
    mesh=_sc_mesh,
    scratch_types=[
        pltpu.VMEM((NCH, CH), jnp.int32),
        pltpu.VMEM((NCH, CH), jnp.int32),
    ] + [pltpu.VMEM((CH, D_FEAT), jnp.float32)] * NBUF + [
        pltpu.VMEM_SHARED((N_LOC, D_FEAT), jnp.float32),
    ] + [pltpu.SemaphoreType.DMA] * NBUF,
)(_edge_body)


ROWS_BLK = 1000
N_BLKS = N_NODES // ROWS_BLK


def _hnorm_body(h_ref, d_ref, o_ref):
    o_ref[...] = h_ref[...] / d_ref[...]


def _hnorm(hidden, degree):
    return pl.pallas_call(
        _hnorm_body,
        out_shape=jax.ShapeDtypeStruct((N_NODES, D_FEAT), jnp.float32),
        grid=(N_BLKS,),
        in_specs=[
            pl.BlockSpec((ROWS_BLK, D_FEAT), lambda i: (i, 0)),
            pl.BlockSpec((ROWS_BLK, 1), lambda i: (i, 0)),
        ],
        out_specs=pl.BlockSpec((ROWS_BLK, D_FEAT), lambda i: (i, 0)),
    )(hidden, degree)


APPLY_BLK = 1000
APPLY_PER_CORE = HALF // APPLY_BLK
N_APPLY_BLKS = N_NODES // APPLY_BLK


def _apply_body(stack_ref, p_ref, h_ref, wt_ref, b_ref, o_ref):
    red = p_ref[0] + h_ref[...]
    o_ref[:, :D_FEAT] = stack_ref[...]
    o_ref[:, D_FEAT:] = (
        jnp.dot(red, wt_ref[...], preferred_element_type=jnp.float32)
        + b_ref[...]
    )


def _apply(stack, partials, hidden, wt, b2):
    return pl.pallas_call(
        _apply_body,
        out_shape=jax.ShapeDtypeStruct((N_NODES, 2 * D_FEAT), jnp.float32),
        grid=(N_APPLY_BLKS,),
        in_specs=[
            pl.BlockSpec((APPLY_BLK, D_FEAT), lambda i: (i, 0)),
            pl.BlockSpec(
                (1, APPLY_BLK, D_FEAT),
                lambda i: (i // APPLY_PER_CORE, i % APPLY_PER_CORE, 0),
            ),
            pl.BlockSpec((APPLY_BLK, D_FEAT), lambda i: (i, 0)),
            pl.BlockSpec((D_FEAT, D_FEAT), lambda i: (0, 0)),
            pl.BlockSpec((1, D_FEAT), lambda i: (0, 0)),
        ],
        out_specs=pl.BlockSpec((APPLY_BLK, 2 * D_FEAT), lambda i: (i, 0)),
    )(stack, partials, hidden, wt, b2)


def kernel(hidden, degree, stack, W, b, edge_index):
    hnorm = _hnorm(hidden, degree)
    src = edge_index[0]
    dst = edge_index[1]
    trash = HALF + jnp.arange(N_EDGES, dtype=jnp.int32) % (N_LOC - HALF)
    d0 = jnp.where(dst < HALF, dst, trash)
    d1 = jnp.where(dst >= HALF, dst - HALF, trash)
    srcr = src.reshape(NS, NCH, CH)
    dstr = jnp.stack([d0, d1]).reshape(NC, NS, NCH, CH)
    partials = _edge_call(hnorm, srcr, dstr)
    return _apply(stack, partials, hidden, W.T, b.reshape(1, D_FEAT))


# speedup vs baseline: 1.3824x; 1.0104x over previous
import functools

import jax
import jax.numpy as jnp
from jax import lax
from jax.experimental import pallas as pl
from jax.experimental.pallas import tpu as pltpu
from jax.experimental.pallas import tpu_sc as plsc

N_NODES = 10000
N_EDGES = 320000
D_FEAT = 128

NC = 2
NS = 16
HALF = N_NODES // NC
CH = 80
NCH = 250
E_PER_S = NCH * CH
N_LOC = 5120
RPT = N_LOC // NS
ZB = 64
NBUF = 2

_sc_mesh = plsc.VectorSubcoreMesh(core_axis_name="c", subcore_axis_name="s")


def _edge_body(hnorm_hbm, srcr_hbm, dstr_hbm, out_hbm,
               src_v, dst_v, *rest):
    rows = rest[:NBUF]
    r0 = rows[0]
    acc = rest[NBUF]
    gsems = rest[NBUF + 1:]
    cid = lax.axis_index("c")
    sid = lax.axis_index("s")

    zv = jnp.zeros((16,), jnp.float32)

    def zbody(i, _):
        for k in range(D_FEAT // 16):
            r0[i, pl.ds(k * 16, 16)] = zv
        return 0

    lax.fori_loop(0, ZB, zbody, 0)
    for k in range(RPT // ZB):
        pltpu.sync_copy(r0.at[pl.ds(0, ZB)],
                        acc.at[pl.ds(sid * RPT + k * ZB, ZB)])

    pltpu.sync_copy(srcr_hbm.at[sid], src_v)
    pltpu.sync_copy(dstr_hbm.at[cid, sid], dst_v)
    plsc.subcore_barrier()

    for k in range(NBUF):
        pltpu.async_copy(hnorm_hbm.at[src_v.at[k]], rows[k], gsems[k])

    def body(jj, _):
        for k in range(NBUF):
            j = NBUF * jj + k
            pltpu.make_async_copy(
                hnorm_hbm.at[src_v.at[j]], rows[k], gsems[k]).wait()
            pltpu.sync_copy(rows[k], acc.at[dst_v.at[j]], add=True)

            @pl.when(j + NBUF < NCH)
            def _():
                pltpu.async_copy(
                    hnorm_hbm.at[src_v.at[j + NBUF]], rows[k], gsems[k])

        return 0

    lax.fori_loop(0, NCH // NBUF, body, 0)

    plsc.subcore_barrier()
    pltpu.sync_copy(acc.at[pl.ds(sid * RPT, RPT)],
                    out_hbm.at[cid, pl.ds(sid * RPT, RPT)])


_edge_call = functools.partial(
    pl.kernel,
    out_type=jax.ShapeDtypeStruct((NC, N_LOC, D_FEAT), jnp.float32),
    mesh=_sc_mesh,
    scratch_types=[
        pltpu.VMEM((NCH, CH), jnp.int32),
        pltpu.VMEM((NCH, CH), jnp.int32),
    ] + [pltpu.VMEM((CH, D_FEAT), jnp.float32)] * NBUF + [
        pltpu.VMEM_SHARED((N_LOC, D_FEAT), jnp.float32),
    ] + [pltpu.SemaphoreType.DMA] * NBUF,
)(_edge_body)


ROWS_BLK = 2000
N_BLKS = N_NODES // ROWS_BLK


def _hnorm_body(h_ref, d_ref, o_ref):
    o_ref[...] = h_ref[...] / d_ref[...]


def _hnorm(hidden, degree):
    return pl.pallas_call(
        _hnorm_body,
        out_shape=jax.ShapeDtypeStruct((N_NODES, D_FEAT), jnp.float32),
        grid=(N_BLKS,),
        in_specs=[
            pl.BlockSpec((ROWS_BLK, D_FEAT), lambda i: (i, 0)),
            pl.BlockSpec((ROWS_BLK, 1), lambda i: (i, 0)),
        ],
        out_specs=pl.BlockSpec((ROWS_BLK, D_FEAT), lambda i: (i, 0)),
    )(hidden, degree)


APPLY_BLK = 1000
APPLY_PER_CORE = HALF // APPLY_BLK
N_APPLY_BLKS = N_NODES // APPLY_BLK


def _apply_body(stack_ref, p_ref, h_ref, wt_ref, b_ref, o_ref):
    red = p_ref[0] + h_ref[...]
    o_ref[:, :D_FEAT] = stack_ref[...]
    o_ref[:, D_FEAT:] = (
        jnp.dot(red, wt_ref[...], preferred_element_type=jnp.float32)
        + b_ref[...]
    )


def _apply(stack, partials, hidden, wt, b2):
    return pl.pallas_call(
        _apply_body,
        out_shape=jax.ShapeDtypeStruct((N_NODES, 2 * D_FEAT), jnp.float32),
        grid=(N_APPLY_BLKS,),
        in_specs=[
            pl.BlockSpec((APPLY_BLK, D_FEAT), lambda i: (i, 0)),
            pl.BlockSpec(
                (1, APPLY_BLK, D_FEAT),
                lambda i: (i // APPLY_PER_CORE, i % APPLY_PER_CORE, 0),
            ),
            pl.BlockSpec((APPLY_BLK, D_FEAT), lambda i: (i, 0)),
            pl.BlockSpec((D_FEAT, D_FEAT), lambda i: (0, 0)),
            pl.BlockSpec((1, D_FEAT), lambda i: (0, 0)),
        ],
        out_specs=pl.BlockSpec((APPLY_BLK, 2 * D_FEAT), lambda i: (i, 0)),
    )(stack, partials, hidden, wt, b2)


def kernel(hidden, degree, stack, W, b, edge_index):
    hnorm = _hnorm(hidden, degree)
    src = edge_index[0]
    dst = edge_index[1]
    trash = HALF + jnp.arange(N_EDGES, dtype=jnp.int32) % (N_LOC - HALF)
    d0 = jnp.where(dst < HALF, dst, trash)
    d1 = jnp.where(dst >= HALF, dst - HALF, trash)
    srcr = src.reshape(NS, NCH, CH)
    dstr = jnp.stack([d0, d1]).reshape(NC, NS, NCH, CH)
    partials = _edge_call(hnorm, srcr, dstr)
    return _apply(stack, partials, hidden, W.T, b.reshape(1, D_FEAT))
